# Initial kernel scaffold; baseline (speedup 1.0000x reference)
import jax
import jax.numpy as jnp
from jax.experimental import pallas as pl


def kernel(v, params):
    raise NotImplementedError("write your pallas kernel here")


# speedup vs baseline: 64.1059x; 64.1059x over previous
import functools

import jax
import jax.numpy as jnp
from jax.experimental import pallas as pl
from jax.experimental.pallas import tpu as pltpu

D = 128
HEAD = 8
HEAD_DIM = 16
KATT = 7
KCONV = 3
KK = KCONV * KCONV
B = 16
L = 1024
NBL = B * L
VOCAB_PAD = 32


def _col(x, j):
    return jnp.broadcast_to(x[:, j:j + 1], x.shape)


def _shift_reflect(x, d):
    if d == 0:
        return x
    n = x.shape[1]
    li = jax.lax.broadcasted_iota(jnp.int32, x.shape, 1)
    if d > 0:
        y = jnp.concatenate([x[:, d:], x[:, :d]], axis=1)
        for t in range(d):
            y = jnp.where(li == (n - d + t), _col(x, n - 2 - t), y)
    else:
        dd = -d
        y = jnp.concatenate([x[:, n - dd:], x[:, :n - dd]], axis=1)
        for t in range(dd):
            y = jnp.where(li == t, _col(x, dd - t), y)
    return y


def _shift_zero(x, d):
    if d == 0:
        return x
    n = x.shape[1]
    li = jax.lax.broadcasted_iota(jnp.int32, x.shape, 1)
    if d > 0:
        y = jnp.concatenate([x[:, d:], x[:, :d]], axis=1)
        return jnp.where(li >= n - d, 0.0, y)
    dd = -d
    y = jnp.concatenate([x[:, n - dd:], x[:, :n - dd]], axis=1)
    return jnp.where(li < dd, 0.0, y)


def _acmix(x, wqkv_ref, bqkv_ref, cpw_ref, cpb_ref, wfc_ref, wcat_ref,
           depb_ref, r1_ref):
    f32 = jnp.float32
    qkv = jnp.dot(wqkv_ref[...], x, preferred_element_type=f32) + bqkv_ref[...]
    q = qkv[0:D]
    k = qkv[D:2 * D]
    v = qkv[2 * D:3 * D]

    lin = (jax.lax.broadcasted_iota(jnp.int32, (HEAD_DIM, L), 1).astype(f32)
           * (2.0 / (L - 1)) - 1.0)
    pe = cpw_ref[:, 0:1] * lin - cpw_ref[:, 1:2] + cpb_ref[...]
    pe_full = jnp.concatenate([pe] * HEAD, axis=0)

    kp = k - pe_full
    qs = q * (float(HEAD_DIM) ** -0.5)

    att = []
    for w in range(KATT):
        ks = _shift_reflect(kp, w - (KATT // 2))
        att.append(jnp.sum((qs * ks).reshape(HEAD, HEAD_DIM, L), axis=1))
    atts = jnp.stack(att, axis=0)
    m = jnp.max(atts, axis=0)
    e = jnp.exp(atts - m[None])
    rden = r1_ref[0, 0] / jnp.sum(e, axis=0)

    oa = jnp.zeros((HEAD, HEAD_DIM, L), f32)
    for w in range(KATT):
        vs = _shift_reflect(v, w - (KATT // 2)).reshape(HEAD, HEAD_DIM, L)
        oa = oa + (e[w] * rden)[:, None, :] * vs
    out_att = oa.reshape(D, L)

    f144 = jnp.dot(wfc_ref[...], qkv, preferred_element_type=f32)
    fcat = jnp.concatenate(
        [_shift_zero(f144, -1), f144, _shift_zero(f144, 1)], axis=0)
    out_conv = jnp.dot(wcat_ref[...], fcat, preferred_element_type=f32)
    out_conv = out_conv + depb_ref[...]

    return jnp.maximum(out_att + out_conv, 0.0)


def _bn_apply(y, stp_ref, g_ref, bb_ref):
    s = jnp.sum(stp_ref[...], axis=0)
    mean = s[:, 0:1] * (1.0 / NBL)
    var = s[:, 1:2] * (1.0 / NBL) - mean * mean
    inv = jax.lax.rsqrt(var + 1e-5)
    return (y - mean) * (inv * g_ref[...]) + bb_ref[...]


def _emit_stats(y, st_ref):
    s1 = jnp.sum(y, axis=1, keepdims=True)
    s2 = jnp.sum(y * y, axis=1, keepdims=True)
    st_ref[...] = jnp.concatenate([s1, s2], axis=1)[None]


def _layer0_body(v_ref, embt_ref, wqkv_ref, bqkv_ref, cpw_ref, cpb_ref,
                 wfc_ref, wcat_ref, depb_ref, r1_ref, y_ref, st_ref):
    vv = jnp.clip(v_ref[0], 0, 25)
    oh = (jax.lax.broadcasted_iota(jnp.int32, (VOCAB_PAD, L), 0) == vv)
    x = jnp.dot(embt_ref[...], oh.astype(jnp.float32),
                preferred_element_type=jnp.float32)
    y = _acmix(x, wqkv_ref, bqkv_ref, cpw_ref, cpb_ref, wfc_ref, wcat_ref,
               depb_ref, r1_ref)
    y_ref[...] = y
    _emit_stats(y, st_ref)


def _layern_body(yin_ref, stp_ref, g_ref, bb_ref, wqkv_ref, bqkv_ref,
                 cpw_ref, cpb_ref, wfc_ref, wcat_ref, depb_ref, r1_ref,
                 y_ref, st_ref):
    x = _bn_apply(yin_ref[...], stp_ref, g_ref, bb_ref)
    y = _acmix(x, wqkv_ref, bqkv_ref, cpw_ref, cpb_ref, wfc_ref, wcat_ref,
               depb_ref, r1_ref)
    y_ref[...] = y
    _emit_stats(y, st_ref)


def _final_body(yin_ref, stp_ref, g_ref, bb_ref, out_ref):
    out_ref[...] = _bn_apply(yin_ref[...], stp_ref, g_ref, bb_ref)


def _full(shape):
    return pl.BlockSpec(shape, lambda b: tuple(0 for _ in shape))


_SLAB = pl.BlockSpec((D, L), lambda b: (b, 0))
_STATS_OUT = pl.BlockSpec((1, D, 2), lambda b: (b, 0, 0))
_WEIGHT_SPECS = [
    _full((3 * D, D)),
    _full((3 * D, 1)),
    _full((HEAD_DIM, 2)),
    _full((HEAD_DIM, 1)),
    _full((KK * HEAD_DIM, 3 * D)),
    _full((D, KCONV * KK * HEAD_DIM)),
    _full((D, 1)),
    pl.BlockSpec(memory_space=pltpu.SMEM),
]
_COMPILER_PARAMS = pltpu.CompilerParams(dimension_semantics=("parallel",))


def _prep_layer(p):
    f32 = jnp.float32
    wqkv = jnp.concatenate([p['conv1_w'], p['conv2_w'], p['conv3_w']], axis=0)
    bqkv = jnp.concatenate([p['conv1_b'], p['conv2_b'], p['conv3_b']])[:, None]
    eye = jnp.eye(HEAD_DIM, dtype=f32)
    wfc = jnp.kron(p['fc_w'].astype(f32), eye)
    wg = p['dep_w'].reshape(HEAD_DIM, D // HEAD_DIM, KK, KCONV)
    wcat = jnp.einsum('goct,gh->gotch', wg, eye).reshape(
        D, KCONV * KK * HEAD_DIM) * p['rate2']
    depb = (p['dep_b'] * p['rate2'])[:, None]
    r1 = p['rate1'].reshape(1, 1)
    return (wqkv, bqkv, p['conv_p_w'], p['conv_p_b'][:, None], wfc, wcat,
            depb, r1)


def kernel(v, params):
    f32 = jnp.float32
    w = [_prep_layer(params['layer%d' % i]) for i in range(3)]
    embt = jnp.zeros((D, VOCAB_PAD), f32).at[:, :26].set(params['emb'].T)
    v3 = v.astype(jnp.int32).reshape(B, 1, L)

    y_shape = jax.ShapeDtypeStruct((B * D, L), f32)
    st_shape = jax.ShapeDtypeStruct((B, D, 2), f32)

    y0, s0 = pl.pallas_call(
        _layer0_body,
        grid=(B,),
        in_specs=[
            pl.BlockSpec((1, 1, L), lambda b: (b, 0, 0)),
            _full((D, VOCAB_PAD)),
        ] + _WEIGHT_SPECS,
        out_specs=[_SLAB, _STATS_OUT],
        out_shape=[y_shape, st_shape],
        compiler_params=_COMPILER_PARAMS,
    )(v3, embt, *w[0])

    def norm_layer(y, s, lp, wts):
        return pl.pallas_call(
            _layern_body,
            grid=(B,),
            in_specs=[
                _SLAB,
                _full((B, D, 2)),
                _full((D, 1)),
                _full((D, 1)),
            ] + _WEIGHT_SPECS,
            out_specs=[_SLAB, _STATS_OUT],
            out_shape=[y_shape, st_shape],
            compiler_params=_COMPILER_PARAMS,
        )(y, s, lp['bn_g'][:, None], lp['bn_b'][:, None], *wts)

    y1, s1 = norm_layer(y0, s0, params['layer0'], w[1])
    y2, s2 = norm_layer(y1, s1, params['layer1'], w[2])

    lp2 = params['layer2']
    out = pl.pallas_call(
        _final_body,
        grid=(B,),
        in_specs=[_SLAB, _full((B, D, 2)), _full((D, 1)), _full((D, 1))],
        out_specs=_SLAB,
        out_shape=y_shape,
        compiler_params=_COMPILER_PARAMS,
    )(y2, s2, lp2['bn_g'][:, None], lp2['bn_b'][:, None])

    return out.reshape(B, L, D)
